# Initial kernel scaffold; baseline (speedup 1.0000x reference)
import jax
import jax.numpy as jnp
from jax.experimental import pallas as pl


def kernel(x, params):
    raise NotImplementedError("write your pallas kernel here")


# speedup vs baseline: 1.3140x; 1.3140x over previous
import functools

import jax
import jax.numpy as jnp
from jax import lax
from jax.experimental import pallas as pl
from jax.experimental.pallas import tpu as pltpu
from jax.experimental.pallas import tpu_sc as plsc

_B = 128
_P = 1024
_E = 8
_L = 16


def _im2col(x):
    pos = lax.broadcasted_iota(jnp.int32, (1, _P), 1)
    hh = pos // 32
    ww = pos % 32
    parts = []
    for dh in (-1, 0, 1):
        for dw in (-1, 0, 1):
            s = 32 * dh + dw
            xs = jnp.roll(x, -s, axis=1) if s else x
            conds = []
            if dh == 1:
                conds.append(hh <= 30)
            if dh == -1:
                conds.append(hh >= 1)
            if dw == 1:
                conds.append(ww <= 30)
            if dw == -1:
                conds.append(ww >= 1)
            if conds:
                m = conds[0]
                for c in conds[1:]:
                    m = m & c
                xs = jnp.where(m, xs, 0.0)
            parts.append(xs)
    return jnp.concatenate(parts, axis=0)


def _flat_w(w):
    co, ci, _, _ = w.shape
    return w.transpose(0, 2, 3, 1).reshape(co, 9 * ci)


def _make_router(cin, cout, interpret=False):
    k9 = 9 * cin

    def body(x_ref, wc_ref, bc_ref, dw_ref, db_ref, out_ref):
        xcol = _im2col(x_ref[0])
        r = jnp.maximum(
            jnp.dot(wc_ref[...], xcol, preferred_element_type=jnp.float32, precision=lax.Precision.HIGHEST)
            + bc_ref[...], 0.0)
        pooled = jnp.sum(r, axis=1, keepdims=True) * (1.0 / _P)
        logits = jnp.sum(dw_ref[...] * pooled, axis=0, keepdims=True) + db_ref[...]
        out_ref[0] = logits

    return pl.pallas_call(
        body,
        grid=(_B,),
        in_specs=[
            pl.BlockSpec((1, cin, _P), lambda b: (b, 0, 0)),
            pl.BlockSpec((cout, k9), lambda b: (0, 0)),
            pl.BlockSpec((cout, 1), lambda b: (0, 0)),
            pl.BlockSpec((cout, _E), lambda b: (0, 0)),
            pl.BlockSpec((1, _E), lambda b: (0, 0)),
        ],
        out_specs=pl.BlockSpec((1, 1, _E), lambda b: (b, 0, 0)),
        out_shape=jax.ShapeDtypeStruct((_B, 1, _E), jnp.float32),
        interpret=interpret,
    )


@functools.cache
def _make_route_sc():
    @functools.partial(
        pl.kernel,
        out_type=[
            jax.ShapeDtypeStruct((_B,), jnp.int32),
            jax.ShapeDtypeStruct((_B,), jnp.int32),
            jax.ShapeDtypeStruct((_B,), jnp.float32),
            jax.ShapeDtypeStruct((_B,), jnp.float32),
        ],
        mesh=plsc.VectorSubcoreMesh(core_axis_name="c", subcore_axis_name="s"),
        scratch_types=[
            pltpu.VMEM((_E, _B), jnp.float32),
            pltpu.VMEM((_L,), jnp.int32),
            pltpu.VMEM((_L,), jnp.int32),
            pltpu.VMEM((_L,), jnp.float32),
            pltpu.VMEM((_L,), jnp.float32),
        ])
    def _route_sc(lt_hbm, i0_hbm, i1_hbm, g0_hbm, g1_hbm,
                  lt_v, i0_v, i1_v, g0_v, g1_v):
        n_groups = _B // _L
        wid = lax.axis_index("s") * 2 + lax.axis_index("c")

        @pl.when(wid < n_groups)
        def _():
            pltpu.sync_copy(lt_hbm, lt_v)
            base = wid * _L
            v = [lt_v[e, pl.ds(base, _L)] for e in range(_E)]
            best1 = v[0]
            bi1 = jnp.zeros((_L,), jnp.int32)
            for e in range(1, _E):
                m = v[e] > best1
                best1 = jnp.where(m, v[e], best1)
                bi1 = jnp.where(m, e, bi1)
            best2 = jnp.full((_L,), -3.0e38, jnp.float32)
            bi2 = jnp.zeros((_L,), jnp.int32)
            for e in range(_E):
                m = (bi1 != e) & (v[e] > best2)
                best2 = jnp.where(m, v[e], best2)
                bi2 = jnp.where(m, e, bi2)
            ga = 1.0 / (1.0 + jnp.exp(best2 - best1))
            i0_v[...] = bi1
            i1_v[...] = bi2
            g0_v[...] = ga
            g1_v[...] = 1.0 - ga
            pltpu.sync_copy(i0_v, i0_hbm.at[pl.ds(base, _L)])
            pltpu.sync_copy(i1_v, i1_hbm.at[pl.ds(base, _L)])
            pltpu.sync_copy(g0_v, g0_hbm.at[pl.ds(base, _L)])
            pltpu.sync_copy(g1_v, g1_hbm.at[pl.ds(base, _L)])

    return _route_sc


def _make_experts(cin, cout, pooled, interpret=False):
    k9 = 9 * cin
    k9h = 9 * cout

    def body(i0_ref, i1_ref, g0_ref, g1_ref, x_ref,
             w1_ref, b1_ref, w2_ref, b2_ref, out_ref):
        b = pl.program_id(0)
        e0 = i0_ref[b]
        e1 = i1_ref[b]
        g0 = g0_ref[b]
        g1 = g1_ref[b]
        xcol = _im2col(x_ref[0])
        w1p = jnp.concatenate([w1_ref[e0], w1_ref[e1]], axis=0)
        b1p = jnp.concatenate([b1_ref[e0], b1_ref[e1]], axis=0)
        hh = jnp.maximum(
            jnp.dot(w1p, xcol, preferred_element_type=jnp.float32, precision=lax.Precision.HIGHEST) + b1p, 0.0)
        y = None
        for sl, e, g in ((slice(0, cout), e0, g0),
                         (slice(cout, 2 * cout), e1, g1)):
            hcol = _im2col(hh[sl])
            ye = jnp.maximum(
                jnp.dot(w2_ref[e], hcol, preferred_element_type=jnp.float32, precision=lax.Precision.HIGHEST)
                + b2_ref[e], 0.0)
            ye = g * ye
            y = ye if y is None else y + ye
        if pooled:
            out_ref[0] = jnp.sum(y, axis=1, keepdims=True) * (1.0 / _P)
        else:
            out_ref[0] = y

    out_shape = (jax.ShapeDtypeStruct((_B, cout, 1), jnp.float32) if pooled
                 else jax.ShapeDtypeStruct((_B, cout, _P), jnp.float32))
    out_block = (1, cout, 1) if pooled else (1, cout, _P)
    return pl.pallas_call(
        body,
        grid=(_B,),
        in_specs=[
            pl.BlockSpec(memory_space=pltpu.SMEM),
            pl.BlockSpec(memory_space=pltpu.SMEM),
            pl.BlockSpec(memory_space=pltpu.SMEM),
            pl.BlockSpec(memory_space=pltpu.SMEM),
            pl.BlockSpec((1, cin, _P), lambda b: (b, 0, 0)),
            pl.BlockSpec((_E, cout, k9), lambda b: (0, 0, 0)),
            pl.BlockSpec((_E, cout, 1), lambda b: (0, 0, 0)),
            pl.BlockSpec((_E, cout, k9h), lambda b: (0, 0, 0)),
            pl.BlockSpec((_E, cout, 1), lambda b: (0, 0, 0)),
        ],
        out_specs=pl.BlockSpec(out_block, lambda b: (b, 0, 0)),
        out_shape=out_shape,
        interpret=interpret,
    )


def _make_head(n_out, interpret=False):
    def body(p_ref, w_ref, b_ref, out_ref):
        out_ref[...] = (
            jnp.dot(p_ref[...], w_ref[...], preferred_element_type=jnp.float32, precision=lax.Precision.HIGHEST)
            + b_ref[...])

    return pl.pallas_call(
        body,
        out_shape=jax.ShapeDtypeStruct((_B, n_out), jnp.float32),
        interpret=interpret,
    )


def kernel(x, params):
    h = x.reshape(_B, x.shape[1], _P)
    blocks = params['blocks']
    n = len(blocks)
    for bi, blk in enumerate(blocks):
        cin = h.shape[1]
        rt = blk['router']
        cout = rt['cw'].shape[0]
        router = _make_router(cin, cout)
        logits = router(h, _flat_w(rt['cw']), rt['cb'].reshape(cout, 1),
                        rt['dw'], rt['db'].reshape(1, _E))
        lt = logits.reshape(_B, _E).T
        i0, i1, g0, g1 = _make_route_sc()(lt)
        w1 = jnp.stack([_flat_w(ex['w'][0]) for ex in blk['experts']])
        b1 = jnp.stack([ex['b'][0] for ex in blk['experts']]).reshape(_E, cout, 1)
        w2 = jnp.stack([_flat_w(ex['w'][1]) for ex in blk['experts']])
        b2 = jnp.stack([ex['b'][1] for ex in blk['experts']]).reshape(_E, cout, 1)
        experts = _make_experts(cin, cout, pooled=(bi == n - 1))
        h = experts(i0, i1, g0, g1, h, w1, b1, w2, b2)
    pooled = h.reshape(_B, -1)
    hw = params['head']['w']
    head = _make_head(hw.shape[1])
    return head(pooled, hw, params['head']['b'].reshape(1, -1))


# speedup vs baseline: 1.3506x; 1.0278x over previous
import functools

import jax
import jax.numpy as jnp
from jax import lax
from jax.experimental import pallas as pl
from jax.experimental.pallas import tpu as pltpu
from jax.experimental.pallas import tpu_sc as plsc

_B = 128
_P = 1024
_E = 8
_L = 16


def _im2col(x):
    pos = lax.broadcasted_iota(jnp.int32, (1, _P), 1)
    hh = pos // 32
    ww = pos % 32
    parts = []
    for dh in (-1, 0, 1):
        for dw in (-1, 0, 1):
            s = 32 * dh + dw
            xs = jnp.roll(x, -s, axis=1) if s else x
            conds = []
            if dh == 1:
                conds.append(hh <= 30)
            if dh == -1:
                conds.append(hh >= 1)
            if dw == 1:
                conds.append(ww <= 30)
            if dw == -1:
                conds.append(ww >= 1)
            if conds:
                m = conds[0]
                for c in conds[1:]:
                    m = m & c
                xs = jnp.where(m, xs, 0.0)
            parts.append(xs)
    return jnp.concatenate(parts, axis=0)


def _flat_w(w):
    co, ci, _, _ = w.shape
    return w.transpose(0, 2, 3, 1).reshape(co, 9 * ci)


def _make_router(cin, cout, interpret=False):
    k9 = 9 * cin

    def body(x_ref, wc_ref, bc_ref, dw_ref, db_ref, out_ref):
        xcol = _im2col(x_ref[0])
        r = jnp.maximum(
            jnp.dot(wc_ref[...], xcol, preferred_element_type=jnp.float32, precision=lax.Precision.HIGHEST)
            + bc_ref[...], 0.0)
        pooled = jnp.sum(r, axis=1, keepdims=True) * (1.0 / _P)
        logits = jnp.sum(dw_ref[...] * pooled, axis=0, keepdims=True) + db_ref[...]
        out_ref[0] = logits

    return pl.pallas_call(
        body,
        grid=(_B,),
        in_specs=[
            pl.BlockSpec((1, cin, _P), lambda b: (b, 0, 0)),
            pl.BlockSpec((cout, k9), lambda b: (0, 0)),
            pl.BlockSpec((cout, 1), lambda b: (0, 0)),
            pl.BlockSpec((cout, _E), lambda b: (0, 0)),
            pl.BlockSpec((1, _E), lambda b: (0, 0)),
        ],
        out_specs=pl.BlockSpec((1, 1, _E), lambda b: (b, 0, 0)),
        out_shape=jax.ShapeDtypeStruct((_B, 1, _E), jnp.float32),
        interpret=interpret,
    )


@functools.cache
def _make_route_sc():
    @functools.partial(
        pl.kernel,
        out_type=[
            jax.ShapeDtypeStruct((_B,), jnp.int32),
            jax.ShapeDtypeStruct((_B,), jnp.int32),
            jax.ShapeDtypeStruct((_B,), jnp.float32),
            jax.ShapeDtypeStruct((_B,), jnp.float32),
        ],
        mesh=plsc.VectorSubcoreMesh(core_axis_name="c", subcore_axis_name="s"),
        scratch_types=[
            pltpu.VMEM((_E, _B), jnp.float32),
            pltpu.VMEM((_L,), jnp.int32),
            pltpu.VMEM((_L,), jnp.int32),
            pltpu.VMEM((_L,), jnp.float32),
            pltpu.VMEM((_L,), jnp.float32),
        ])
    def _route_sc(lt_hbm, i0_hbm, i1_hbm, g0_hbm, g1_hbm,
                  lt_v, i0_v, i1_v, g0_v, g1_v):
        n_groups = _B // _L
        wid = lax.axis_index("s") * 2 + lax.axis_index("c")

        @pl.when(wid < n_groups)
        def _():
            pltpu.sync_copy(lt_hbm, lt_v)
            base = wid * _L
            v = [lt_v[e, pl.ds(base, _L)] for e in range(_E)]
            best1 = v[0]
            bi1 = jnp.zeros((_L,), jnp.int32)
            for e in range(1, _E):
                m = v[e] > best1
                best1 = jnp.where(m, v[e], best1)
                bi1 = jnp.where(m, e, bi1)
            best2 = jnp.full((_L,), -3.0e38, jnp.float32)
            bi2 = jnp.zeros((_L,), jnp.int32)
            for e in range(_E):
                m = (bi1 != e) & (v[e] > best2)
                best2 = jnp.where(m, v[e], best2)
                bi2 = jnp.where(m, e, bi2)
            ga = 1.0 / (1.0 + jnp.exp(best2 - best1))
            i0_v[...] = bi1
            i1_v[...] = bi2
            g0_v[...] = ga
            g1_v[...] = 1.0 - ga
            pltpu.sync_copy(i0_v, i0_hbm.at[pl.ds(base, _L)])
            pltpu.sync_copy(i1_v, i1_hbm.at[pl.ds(base, _L)])
            pltpu.sync_copy(g0_v, g0_hbm.at[pl.ds(base, _L)])
            pltpu.sync_copy(g1_v, g1_hbm.at[pl.ds(base, _L)])

    return _route_sc


def _make_experts(cin, cout, tail, interpret=False):
    k9 = 9 * cin
    k9h = 9 * cout
    prec = lax.Precision.HIGHEST

    def body(i0_ref, i1_ref, g0_ref, g1_ref, x_ref,
             w1_ref, b1_ref, w2_ref, b2_ref, ta_ref, tb_ref, tc_ref, td_ref,
             *out_refs):
        b = pl.program_id(0)
        e0 = i0_ref[b]
        e1 = i1_ref[b]
        g0 = g0_ref[b]
        g1 = g1_ref[b]
        xcol = _im2col(x_ref[0])
        w1p = jnp.concatenate([w1_ref[e0], w1_ref[e1]], axis=0)
        b1p = jnp.concatenate([b1_ref[e0], b1_ref[e1]], axis=0)
        hh = jnp.maximum(
            jnp.dot(w1p, xcol, preferred_element_type=jnp.float32,
                    precision=prec) + b1p, 0.0)
        y = None
        for sl, e, g in ((slice(0, cout), e0, g0),
                         (slice(cout, 2 * cout), e1, g1)):
            hcol = _im2col(hh[sl])
            ye = jnp.maximum(
                jnp.dot(w2_ref[e], hcol, preferred_element_type=jnp.float32,
                        precision=prec) + b2_ref[e], 0.0)
            ye = g * ye
            y = ye if y is None else y + ye
        if tail == 'head':
            pooled = jnp.sum(y, axis=1, keepdims=True) * (1.0 / _P)
            logits = jnp.sum(ta_ref[...] * pooled, axis=0, keepdims=True) \
                + tb_ref[...]
            out_refs[0][0] = logits
        else:
            out_refs[0][0] = y

            ycol = _im2col(y)
            r = jnp.maximum(
                jnp.dot(ta_ref[...], ycol, preferred_element_type=jnp.float32,
                        precision=prec) + tb_ref[...], 0.0)
            pooled = jnp.sum(r, axis=1, keepdims=True) * (1.0 / _P)
            nlogits = jnp.sum(tc_ref[...] * pooled, axis=0, keepdims=True) \
                + td_ref[...]
            out_refs[1][0] = nlogits

    if tail == 'head':
        n_out = 10
        tail_specs = [
            pl.BlockSpec((cout, n_out), lambda b: (0, 0)),
            pl.BlockSpec((1, n_out), lambda b: (0, 0)),
            pl.BlockSpec((1, 1), lambda b: (0, 0)),
            pl.BlockSpec((1, 1), lambda b: (0, 0)),
        ]
        out_shape = [jax.ShapeDtypeStruct((_B, 1, n_out), jnp.float32)]
        out_specs = [pl.BlockSpec((1, 1, n_out), lambda b: (b, 0, 0))]
    else:
        ncout = tail
        tail_specs = [
            pl.BlockSpec((ncout, 9 * cout), lambda b: (0, 0)),
            pl.BlockSpec((ncout, 1), lambda b: (0, 0)),
            pl.BlockSpec((ncout, _E), lambda b: (0, 0)),
            pl.BlockSpec((1, _E), lambda b: (0, 0)),
        ]
        out_shape = [jax.ShapeDtypeStruct((_B, cout, _P), jnp.float32),
                     jax.ShapeDtypeStruct((_B, 1, _E), jnp.float32)]
        out_specs = [pl.BlockSpec((1, cout, _P), lambda b: (b, 0, 0)),
                     pl.BlockSpec((1, 1, _E), lambda b: (b, 0, 0))]

    return pl.pallas_call(
        body,
        grid=(_B,),
        in_specs=[
            pl.BlockSpec(memory_space=pltpu.SMEM),
            pl.BlockSpec(memory_space=pltpu.SMEM),
            pl.BlockSpec(memory_space=pltpu.SMEM),
            pl.BlockSpec(memory_space=pltpu.SMEM),
            pl.BlockSpec((1, cin, _P), lambda b: (b, 0, 0)),
            pl.BlockSpec((_E, cout, k9), lambda b: (0, 0, 0)),
            pl.BlockSpec((_E, cout, 1), lambda b: (0, 0, 0)),
            pl.BlockSpec((_E, cout, k9h), lambda b: (0, 0, 0)),
            pl.BlockSpec((_E, cout, 1), lambda b: (0, 0, 0)),
        ] + tail_specs,
        out_specs=out_specs,
        out_shape=out_shape,
        interpret=interpret,
    )


def _make_head(n_out, interpret=False):
    def body(p_ref, w_ref, b_ref, out_ref):
        out_ref[...] = (
            jnp.dot(p_ref[...], w_ref[...], preferred_element_type=jnp.float32, precision=lax.Precision.HIGHEST)
            + b_ref[...])

    return pl.pallas_call(
        body,
        out_shape=jax.ShapeDtypeStruct((_B, n_out), jnp.float32),
        interpret=interpret,
    )


def kernel(x, params):
    h = x.reshape(_B, x.shape[1], _P)
    blocks = params['blocks']
    n = len(blocks)
    rt = blocks[0]['router']
    cout0 = rt['cw'].shape[0]
    router = _make_router(h.shape[1], cout0)
    logits = router(h, _flat_w(rt['cw']), rt['cb'].reshape(cout0, 1),
                    rt['dw'], rt['db'].reshape(1, _E))
    for bi, blk in enumerate(blocks):
        cin = h.shape[1]
        cout = blk['router']['cw'].shape[0]
        i0, i1, g0, g1 = _make_route_sc()(logits.reshape(_B, _E).T)
        w1 = jnp.stack([_flat_w(ex['w'][0]) for ex in blk['experts']])
        b1 = jnp.stack([ex['b'][0] for ex in blk['experts']]).reshape(_E, cout, 1)
        w2 = jnp.stack([_flat_w(ex['w'][1]) for ex in blk['experts']])
        b2 = jnp.stack([ex['b'][1] for ex in blk['experts']]).reshape(_E, cout, 1)
        if bi == n - 1:
            experts = _make_experts(cin, cout, tail='head')
            out = experts(i0, i1, g0, g1, h, w1, b1, w2, b2,
                          params['head']['w'],
                          params['head']['b'].reshape(1, -1),
                          jnp.zeros((1, 1), jnp.float32),
                          jnp.zeros((1, 1), jnp.float32))[0]
            return out.reshape(_B, -1)
        nrt = blocks[bi + 1]['router']
        ncout = nrt['cw'].shape[0]
        experts = _make_experts(cin, cout, tail=ncout)
        h, logits = experts(i0, i1, g0, g1, h, w1, b1, w2, b2,
                            _flat_w(nrt['cw']), nrt['cb'].reshape(ncout, 1),
                            nrt['dw'], nrt['db'].reshape(1, _E))


# speedup vs baseline: 2.2302x; 1.6514x over previous
import functools

import jax
import jax.numpy as jnp
from jax import lax
from jax.experimental import pallas as pl
from jax.experimental.pallas import tpu as pltpu
from jax.experimental.pallas import tpu_sc as plsc

_B = 128
_P = 1024
_E = 8
_L = 16


def _im2col(x):
    pos = lax.broadcasted_iota(jnp.int32, (1, _P), 1)
    hh = pos // 32
    ww = pos % 32
    parts = []
    for dh in (-1, 0, 1):
        for dw in (-1, 0, 1):
            s = 32 * dh + dw
            xs = jnp.roll(x, -s, axis=1) if s else x
            conds = []
            if dh == 1:
                conds.append(hh <= 30)
            if dh == -1:
                conds.append(hh >= 1)
            if dw == 1:
                conds.append(ww <= 30)
            if dw == -1:
                conds.append(ww >= 1)
            if conds:
                m = conds[0]
                for c in conds[1:]:
                    m = m & c
                xs = jnp.where(m, xs, jnp.zeros((), xs.dtype))
            parts.append(xs)
    return jnp.concatenate(parts, axis=0)


def _flat_w(w):
    co, ci, _, _ = w.shape
    return w.transpose(0, 2, 3, 1).reshape(co, 9 * ci)


def _split_hl(x):
    hi = x.astype(jnp.bfloat16)
    lo = (x - hi.astype(jnp.float32)).astype(jnp.bfloat16)
    return hi, lo


def _dot3(whi, wlo, xhi, xlo):
    return (jnp.dot(whi, xhi, preferred_element_type=jnp.float32)
            + jnp.dot(whi, xlo, preferred_element_type=jnp.float32)
            + jnp.dot(wlo, xhi, preferred_element_type=jnp.float32))


def _conv_cols(x):
    xhi, xlo = _split_hl(x)
    return _im2col(xhi), _im2col(xlo)


def _make_router(cin, cout, interpret=False):
    k9 = 9 * cin

    def body(x_ref, wch_ref, wcl_ref, bc_ref, dw_ref, db_ref, out_ref):
        xcolh, xcoll = _conv_cols(x_ref[0])
        r = jnp.maximum(
            _dot3(wch_ref[...], wcl_ref[...], xcolh, xcoll) + bc_ref[...], 0.0)
        pooled = jnp.sum(r, axis=1, keepdims=True) * (1.0 / _P)
        logits = jnp.sum(dw_ref[...] * pooled, axis=0, keepdims=True) + db_ref[...]
        out_ref[0] = logits

    return pl.pallas_call(
        body,
        grid=(_B,),
        in_specs=[
            pl.BlockSpec((1, cin, _P), lambda b: (b, 0, 0)),
            pl.BlockSpec((cout, k9), lambda b: (0, 0)),
            pl.BlockSpec((cout, k9), lambda b: (0, 0)),
            pl.BlockSpec((cout, 1), lambda b: (0, 0)),
            pl.BlockSpec((cout, _E), lambda b: (0, 0)),
            pl.BlockSpec((1, _E), lambda b: (0, 0)),
        ],
        out_specs=pl.BlockSpec((1, 1, _E), lambda b: (b, 0, 0)),
        out_shape=jax.ShapeDtypeStruct((_B, 1, _E), jnp.float32),
        interpret=interpret,
    )


@functools.cache
def _make_route_sc():
    @functools.partial(
        pl.kernel,
        out_type=[
            jax.ShapeDtypeStruct((_B,), jnp.int32),
            jax.ShapeDtypeStruct((_B,), jnp.int32),
            jax.ShapeDtypeStruct((_B,), jnp.float32),
            jax.ShapeDtypeStruct((_B,), jnp.float32),
        ],
        mesh=plsc.VectorSubcoreMesh(core_axis_name="c", subcore_axis_name="s"),
        scratch_types=[
            pltpu.VMEM((_E, _B), jnp.float32),
            pltpu.VMEM((_L,), jnp.int32),
            pltpu.VMEM((_L,), jnp.int32),
            pltpu.VMEM((_L,), jnp.float32),
            pltpu.VMEM((_L,), jnp.float32),
        ])
    def _route_sc(lt_hbm, i0_hbm, i1_hbm, g0_hbm, g1_hbm,
                  lt_v, i0_v, i1_v, g0_v, g1_v):
        n_groups = _B // _L
        wid = lax.axis_index("s") * 2 + lax.axis_index("c")

        @pl.when(wid < n_groups)
        def _():
            pltpu.sync_copy(lt_hbm, lt_v)
            base = wid * _L
            v = [lt_v[e, pl.ds(base, _L)] for e in range(_E)]
            best1 = v[0]
            bi1 = jnp.zeros((_L,), jnp.int32)
            for e in range(1, _E):
                m = v[e] > best1
                best1 = jnp.where(m, v[e], best1)
                bi1 = jnp.where(m, e, bi1)
            best2 = jnp.full((_L,), -3.0e38, jnp.float32)
            bi2 = jnp.zeros((_L,), jnp.int32)
            for e in range(_E):
                m = (bi1 != e) & (v[e] > best2)
                best2 = jnp.where(m, v[e], best2)
                bi2 = jnp.where(m, e, bi2)
            ga = 1.0 / (1.0 + jnp.exp(best2 - best1))
            i0_v[...] = bi1
            i1_v[...] = bi2
            g0_v[...] = ga
            g1_v[...] = 1.0 - ga
            pltpu.sync_copy(i0_v, i0_hbm.at[pl.ds(base, _L)])
            pltpu.sync_copy(i1_v, i1_hbm.at[pl.ds(base, _L)])
            pltpu.sync_copy(g0_v, g0_hbm.at[pl.ds(base, _L)])
            pltpu.sync_copy(g1_v, g1_hbm.at[pl.ds(base, _L)])

    return _route_sc


def _make_experts(cin, cout, tail, interpret=False):
    k9 = 9 * cin
    k9h = 9 * cout

    def body(i0_ref, i1_ref, g0_ref, g1_ref, x_ref,
             w1h_ref, w1l_ref, b1_ref, w2h_ref, w2l_ref, b2_ref, *rest):
        if tail == 'head':
            hw_ref, hb_ref = rest[:2]
            out_refs = rest[2:]
        else:
            tah_ref, tal_ref, tb_ref, tc_ref, td_ref = rest[:5]
            out_refs = rest[5:]
        b = pl.program_id(0)
        e0 = i0_ref[b]
        e1 = i1_ref[b]
        g0 = g0_ref[b]
        g1 = g1_ref[b]
        xcolh, xcoll = _conv_cols(x_ref[0])
        w1ph = jnp.concatenate([w1h_ref[e0], w1h_ref[e1]], axis=0)
        w1pl = jnp.concatenate([w1l_ref[e0], w1l_ref[e1]], axis=0)
        b1p = jnp.concatenate([b1_ref[e0], b1_ref[e1]], axis=0)
        hh = jnp.maximum(_dot3(w1ph, w1pl, xcolh, xcoll) + b1p, 0.0)
        y = None
        for sl, e, g in ((slice(0, cout), e0, g0),
                         (slice(cout, 2 * cout), e1, g1)):
            hcolh, hcoll = _conv_cols(hh[sl])
            ye = jnp.maximum(
                _dot3(w2h_ref[e], w2l_ref[e], hcolh, hcoll) + b2_ref[e], 0.0)
            ye = g * ye
            y = ye if y is None else y + ye
        if tail == 'head':
            pooled = jnp.sum(y, axis=1, keepdims=True) * (1.0 / _P)
            logits = jnp.sum(hw_ref[...] * pooled, axis=0, keepdims=True) \
                + hb_ref[...]
            out_refs[0][0] = logits
        else:
            out_refs[0][0] = y
            ycolh, ycoll = _conv_cols(y)
            r = jnp.maximum(
                _dot3(tah_ref[...], tal_ref[...], ycolh, ycoll)
                + tb_ref[...], 0.0)
            pooled = jnp.sum(r, axis=1, keepdims=True) * (1.0 / _P)
            nlogits = jnp.sum(tc_ref[...] * pooled, axis=0, keepdims=True) \
                + td_ref[...]
            out_refs[1][0] = nlogits

    if tail == 'head':
        n_out = 10
        tail_specs = [
            pl.BlockSpec((cout, n_out), lambda b: (0, 0)),
            pl.BlockSpec((1, n_out), lambda b: (0, 0)),
        ]
        out_shape = [jax.ShapeDtypeStruct((_B, 1, n_out), jnp.float32)]
        out_specs = [pl.BlockSpec((1, 1, n_out), lambda b: (b, 0, 0))]
    else:
        ncout = tail
        tail_specs = [
            pl.BlockSpec((ncout, 9 * cout), lambda b: (0, 0)),
            pl.BlockSpec((ncout, 9 * cout), lambda b: (0, 0)),
            pl.BlockSpec((ncout, 1), lambda b: (0, 0)),
            pl.BlockSpec((ncout, _E), lambda b: (0, 0)),
            pl.BlockSpec((1, _E), lambda b: (0, 0)),
        ]
        out_shape = [jax.ShapeDtypeStruct((_B, cout, _P), jnp.float32),
                     jax.ShapeDtypeStruct((_B, 1, _E), jnp.float32)]
        out_specs = [pl.BlockSpec((1, cout, _P), lambda b: (b, 0, 0)),
                     pl.BlockSpec((1, 1, _E), lambda b: (b, 0, 0))]

    return pl.pallas_call(
        body,
        grid=(_B,),
        in_specs=[
            pl.BlockSpec(memory_space=pltpu.SMEM),
            pl.BlockSpec(memory_space=pltpu.SMEM),
            pl.BlockSpec(memory_space=pltpu.SMEM),
            pl.BlockSpec(memory_space=pltpu.SMEM),
            pl.BlockSpec((1, cin, _P), lambda b: (b, 0, 0)),
            pl.BlockSpec((_E, cout, k9), lambda b: (0, 0, 0)),
            pl.BlockSpec((_E, cout, k9), lambda b: (0, 0, 0)),
            pl.BlockSpec((_E, cout, 1), lambda b: (0, 0, 0)),
            pl.BlockSpec((_E, cout, k9h), lambda b: (0, 0, 0)),
            pl.BlockSpec((_E, cout, k9h), lambda b: (0, 0, 0)),
            pl.BlockSpec((_E, cout, 1), lambda b: (0, 0, 0)),
        ] + tail_specs,
        out_specs=out_specs,
        out_shape=out_shape,
        interpret=interpret,
    )


def kernel(x, params):
    h = x.reshape(_B, x.shape[1], _P)
    blocks = params['blocks']
    n = len(blocks)
    rt = blocks[0]['router']
    cout0 = rt['cw'].shape[0]
    router = _make_router(h.shape[1], cout0)
    wch, wcl = _split_hl(_flat_w(rt['cw']))
    logits = router(h, wch, wcl, rt['cb'].reshape(cout0, 1),
                    rt['dw'], rt['db'].reshape(1, _E))
    for bi, blk in enumerate(blocks):
        cin = h.shape[1]
        cout = blk['router']['cw'].shape[0]
        i0, i1, g0, g1 = _make_route_sc()(logits.reshape(_B, _E).T)
        w1h, w1l = _split_hl(jnp.stack([_flat_w(ex['w'][0])
                                        for ex in blk['experts']]))
        b1 = jnp.stack([ex['b'][0] for ex in blk['experts']]).reshape(_E, cout, 1)
        w2h, w2l = _split_hl(jnp.stack([_flat_w(ex['w'][1])
                                        for ex in blk['experts']]))
        b2 = jnp.stack([ex['b'][1] for ex in blk['experts']]).reshape(_E, cout, 1)
        if bi == n - 1:
            experts = _make_experts(cin, cout, tail='head')
            out = experts(i0, i1, g0, g1, h, w1h, w1l, b1, w2h, w2l, b2,
                          params['head']['w'],
                          params['head']['b'].reshape(1, -1))[0]
            return out.reshape(_B, -1)
        nrt = blocks[bi + 1]['router']
        ncout = nrt['cw'].shape[0]
        nwch, nwcl = _split_hl(_flat_w(nrt['cw']))
        experts = _make_experts(cin, cout, tail=ncout)
        h, logits = experts(i0, i1, g0, g1, h, w1h, w1l, b1, w2h, w2l, b2,
                            nwch, nwcl, nrt['cb'].reshape(ncout, 1),
                            nrt['dw'], nrt['db'].reshape(1, _E))


# speedup vs baseline: 2.3923x; 1.0726x over previous
import functools

import jax
import jax.numpy as jnp
from jax import lax
from jax.experimental import pallas as pl
from jax.experimental.pallas import tpu as pltpu
from jax.experimental.pallas import tpu_sc as plsc

_B = 128
_P = 1024
_E = 8
_L = 16


def _im2col(x):
    pos = lax.broadcasted_iota(jnp.int32, (1, _P), 1)
    hh = pos // 32
    ww = pos % 32
    parts = []
    for dh in (-1, 0, 1):
        for dw in (-1, 0, 1):
            s = 32 * dh + dw
            xs = jnp.roll(x, -s, axis=1) if s else x
            conds = []
            if dh == 1:
                conds.append(hh <= 30)
            if dh == -1:
                conds.append(hh >= 1)
            if dw == 1:
                conds.append(ww <= 30)
            if dw == -1:
                conds.append(ww >= 1)
            if conds:
                m = conds[0]
                for c in conds[1:]:
                    m = m & c
                xs = jnp.where(m, xs, jnp.zeros((), xs.dtype))
            parts.append(xs)
    return jnp.concatenate(parts, axis=0)


def _flat_w(w):
    co, ci, _, _ = w.shape
    return w.transpose(0, 2, 3, 1).reshape(co, 9 * ci)


def _split_hl(x):
    hi = x.astype(jnp.bfloat16)
    lo = (x - hi.astype(jnp.float32)).astype(jnp.bfloat16)
    return hi, lo


def _dot3(whi, wlo, xhi, xlo):
    return (jnp.dot(whi, xhi, preferred_element_type=jnp.float32)
            + jnp.dot(whi, xlo, preferred_element_type=jnp.float32)
            + jnp.dot(wlo, xhi, preferred_element_type=jnp.float32))


def _conv_cols(x):
    xhi, xlo = _split_hl(x)
    return _im2col(xhi), _im2col(xlo)


def _make_router(cin, cout, ns=1, interpret=False):
    k9 = 9 * cin

    def body(x_ref, wch_ref, wcl_ref, bc_ref, dw_ref, db_ref, out_ref):
        for j in range(ns):
            xcolh, xcoll = _conv_cols(x_ref[j])
            r = jnp.maximum(
                _dot3(wch_ref[...], wcl_ref[...], xcolh, xcoll)
                + bc_ref[...], 0.0)
            pooled = jnp.sum(r, axis=1, keepdims=True) * (1.0 / _P)
            out_ref[j] = (jnp.sum(dw_ref[...] * pooled, axis=0, keepdims=True)
                          + db_ref[...])

    return pl.pallas_call(
        body,
        grid=(_B // ns,),
        in_specs=[
            pl.BlockSpec((ns, cin, _P), lambda b: (b, 0, 0)),
            pl.BlockSpec((cout, k9), lambda b: (0, 0)),
            pl.BlockSpec((cout, k9), lambda b: (0, 0)),
            pl.BlockSpec((cout, 1), lambda b: (0, 0)),
            pl.BlockSpec((cout, _E), lambda b: (0, 0)),
            pl.BlockSpec((1, _E), lambda b: (0, 0)),
        ],
        out_specs=pl.BlockSpec((ns, 1, _E), lambda b: (b, 0, 0)),
        out_shape=jax.ShapeDtypeStruct((_B, 1, _E), jnp.float32),
        interpret=interpret,
    )


@functools.cache
def _make_route_sc():
    @functools.partial(
        pl.kernel,
        out_type=[
            jax.ShapeDtypeStruct((_B,), jnp.int32),
            jax.ShapeDtypeStruct((_B,), jnp.int32),
            jax.ShapeDtypeStruct((_B,), jnp.float32),
            jax.ShapeDtypeStruct((_B,), jnp.float32),
        ],
        mesh=plsc.VectorSubcoreMesh(core_axis_name="c", subcore_axis_name="s"),
        scratch_types=[
            pltpu.VMEM((_E, _B), jnp.float32),
            pltpu.VMEM((_L,), jnp.int32),
            pltpu.VMEM((_L,), jnp.int32),
            pltpu.VMEM((_L,), jnp.float32),
            pltpu.VMEM((_L,), jnp.float32),
        ])
    def _route_sc(lt_hbm, i0_hbm, i1_hbm, g0_hbm, g1_hbm,
                  lt_v, i0_v, i1_v, g0_v, g1_v):
        n_groups = _B // _L
        wid = lax.axis_index("s") * 2 + lax.axis_index("c")

        @pl.when(wid < n_groups)
        def _():
            pltpu.sync_copy(lt_hbm, lt_v)
            base = wid * _L
            v = [lt_v[e, pl.ds(base, _L)] for e in range(_E)]
            best1 = v[0]
            bi1 = jnp.zeros((_L,), jnp.int32)
            for e in range(1, _E):
                m = v[e] > best1
                best1 = jnp.where(m, v[e], best1)
                bi1 = jnp.where(m, e, bi1)
            best2 = jnp.full((_L,), -3.0e38, jnp.float32)
            bi2 = jnp.zeros((_L,), jnp.int32)
            for e in range(_E):
                m = (bi1 != e) & (v[e] > best2)
                best2 = jnp.where(m, v[e], best2)
                bi2 = jnp.where(m, e, bi2)
            ga = 1.0 / (1.0 + jnp.exp(best2 - best1))
            i0_v[...] = bi1
            i1_v[...] = bi2
            g0_v[...] = ga
            g1_v[...] = 1.0 - ga
            pltpu.sync_copy(i0_v, i0_hbm.at[pl.ds(base, _L)])
            pltpu.sync_copy(i1_v, i1_hbm.at[pl.ds(base, _L)])
            pltpu.sync_copy(g0_v, g0_hbm.at[pl.ds(base, _L)])
            pltpu.sync_copy(g1_v, g1_hbm.at[pl.ds(base, _L)])

    return _route_sc


def _make_experts(cin, cout, tail, ns=1, interpret=False):
    k9 = 9 * cin
    k9h = 9 * cout

    def body(i0_ref, i1_ref, g0_ref, g1_ref, x_ref,
             w1h_ref, w1l_ref, b1_ref, w2h_ref, w2l_ref, b2_ref, *rest):
        if tail == 'head':
            hw_ref, hb_ref = rest[:2]
            out_refs = rest[2:]
        else:
            tah_ref, tal_ref, tb_ref, tc_ref, td_ref = rest[:5]
            out_refs = rest[5:]
        b = pl.program_id(0)
        for j in range(ns):
            s = b * ns + j
            e0 = i0_ref[s]
            e1 = i1_ref[s]
            g0 = g0_ref[s]
            g1 = g1_ref[s]
            xcolh, xcoll = _conv_cols(x_ref[j])
            w1ph = jnp.concatenate([w1h_ref[e0], w1h_ref[e1]], axis=0)
            w1pl = jnp.concatenate([w1l_ref[e0], w1l_ref[e1]], axis=0)
            b1p = jnp.concatenate([b1_ref[e0], b1_ref[e1]], axis=0)
            hh = jnp.maximum(_dot3(w1ph, w1pl, xcolh, xcoll) + b1p, 0.0)
            y = None
            for sl, e, g in ((slice(0, cout), e0, g0),
                             (slice(cout, 2 * cout), e1, g1)):
                hcolh, hcoll = _conv_cols(hh[sl])
                ye = jnp.maximum(
                    _dot3(w2h_ref[e], w2l_ref[e], hcolh, hcoll)
                    + b2_ref[e], 0.0)
                ye = g * ye
                y = ye if y is None else y + ye
            if tail == 'head':
                pooled = jnp.sum(y, axis=1, keepdims=True) * (1.0 / _P)
                out_refs[0][j] = (jnp.sum(hw_ref[...] * pooled, axis=0,
                                          keepdims=True) + hb_ref[...])
            else:
                out_refs[0][j] = y
                ycolh, ycoll = _conv_cols(y)
                r = jnp.maximum(
                    _dot3(tah_ref[...], tal_ref[...], ycolh, ycoll)
                    + tb_ref[...], 0.0)
                pooled = jnp.sum(r, axis=1, keepdims=True) * (1.0 / _P)
                out_refs[1][j] = (jnp.sum(tc_ref[...] * pooled, axis=0,
                                          keepdims=True) + td_ref[...])

    if tail == 'head':
        n_out = 10
        tail_specs = [
            pl.BlockSpec((cout, n_out), lambda b: (0, 0)),
            pl.BlockSpec((1, n_out), lambda b: (0, 0)),
        ]
        out_shape = [jax.ShapeDtypeStruct((_B, 1, n_out), jnp.float32)]
        out_specs = [pl.BlockSpec((ns, 1, n_out), lambda b: (b, 0, 0))]
    else:
        ncout = tail
        tail_specs = [
            pl.BlockSpec((ncout, 9 * cout), lambda b: (0, 0)),
            pl.BlockSpec((ncout, 9 * cout), lambda b: (0, 0)),
            pl.BlockSpec((ncout, 1), lambda b: (0, 0)),
            pl.BlockSpec((ncout, _E), lambda b: (0, 0)),
            pl.BlockSpec((1, _E), lambda b: (0, 0)),
        ]
        out_shape = [jax.ShapeDtypeStruct((_B, cout, _P), jnp.float32),
                     jax.ShapeDtypeStruct((_B, 1, _E), jnp.float32)]
        out_specs = [pl.BlockSpec((ns, cout, _P), lambda b: (b, 0, 0)),
                     pl.BlockSpec((ns, 1, _E), lambda b: (b, 0, 0))]

    return pl.pallas_call(
        body,
        grid=(_B // ns,),
        in_specs=[
            pl.BlockSpec(memory_space=pltpu.SMEM),
            pl.BlockSpec(memory_space=pltpu.SMEM),
            pl.BlockSpec(memory_space=pltpu.SMEM),
            pl.BlockSpec(memory_space=pltpu.SMEM),
            pl.BlockSpec((ns, cin, _P), lambda b: (b, 0, 0)),
            pl.BlockSpec((_E, cout, k9), lambda b: (0, 0, 0)),
            pl.BlockSpec((_E, cout, k9), lambda b: (0, 0, 0)),
            pl.BlockSpec((_E, cout, 1), lambda b: (0, 0, 0)),
            pl.BlockSpec((_E, cout, k9h), lambda b: (0, 0, 0)),
            pl.BlockSpec((_E, cout, k9h), lambda b: (0, 0, 0)),
            pl.BlockSpec((_E, cout, 1), lambda b: (0, 0, 0)),
        ] + tail_specs,
        out_specs=out_specs,
        out_shape=out_shape,
        interpret=interpret,
    )


def _pad8(c):
    return (c + 7) // 8 * 8


def _pad_w(w, co_p, ci_p):
    return jnp.pad(w, ((0, co_p - w.shape[0]), (0, ci_p - w.shape[1]),
                       (0, 0), (0, 0)))


def _pad_rows(a, rows):
    return jnp.pad(a, ((0, rows - a.shape[0]),) + ((0, 0),) * (a.ndim - 1))


def kernel(x, params):
    blocks = params['blocks']
    n = len(blocks)
    cin0 = x.shape[1]
    ci_p = _pad8(cin0)
    h = jnp.pad(x.reshape(_B, cin0, _P), ((0, 0), (0, ci_p - cin0), (0, 0)))
    rt = blocks[0]['router']
    co0_p = _pad8(rt['cw'].shape[0])
    router = _make_router(ci_p, co0_p, ns=8)
    wch, wcl = _split_hl(_flat_w(_pad_w(rt['cw'], co0_p, ci_p)))
    logits = router(h, wch, wcl,
                    _pad_rows(rt['cb'].reshape(-1, 1), co0_p),
                    _pad_rows(rt['dw'], co0_p), rt['db'].reshape(1, _E))
    for bi, blk in enumerate(blocks):
        cout = blk['router']['cw'].shape[0]
        co_p = _pad8(cout)
        i0, i1, g0, g1 = _make_route_sc()(logits.reshape(_B, _E).T)
        w1h, w1l = _split_hl(jnp.stack(
            [_flat_w(_pad_w(ex['w'][0], co_p, ci_p)) for ex in blk['experts']]))
        b1 = jnp.stack([_pad_rows(ex['b'][0].reshape(-1, 1), co_p)
                        for ex in blk['experts']])
        w2h, w2l = _split_hl(jnp.stack(
            [_flat_w(_pad_w(ex['w'][1], co_p, co_p)) for ex in blk['experts']]))
        b2 = jnp.stack([_pad_rows(ex['b'][1].reshape(-1, 1), co_p)
                        for ex in blk['experts']])
        if bi == n - 1:
            experts = _make_experts(ci_p, co_p, tail='head', ns=1)
            out = experts(i0, i1, g0, g1, h, w1h, w1l, b1, w2h, w2l, b2,
                          _pad_rows(params['head']['w'], co_p),
                          params['head']['b'].reshape(1, -1))[0]
            return out.reshape(_B, -1)
        nrt = blocks[bi + 1]['router']
        nco_p = _pad8(nrt['cw'].shape[0])
        nwch, nwcl = _split_hl(_flat_w(_pad_w(nrt['cw'], nco_p, co_p)))
        experts = _make_experts(ci_p, co_p, tail=nco_p,
                                ns=(4 if bi == 0 else 2))
        h, logits = experts(i0, i1, g0, g1, h, w1h, w1l, b1, w2h, w2l, b2,
                            nwch, nwcl,
                            _pad_rows(nrt['cb'].reshape(-1, 1), nco_p),
                            _pad_rows(nrt['dw'], nco_p),
                            nrt['db'].reshape(1, _E))
        ci_p = co_p


# speedup vs baseline: 2.4139x; 1.0090x over previous
import functools

import jax
import jax.numpy as jnp
from jax import lax
from jax.experimental import pallas as pl
from jax.experimental.pallas import tpu as pltpu
from jax.experimental.pallas import tpu_sc as plsc

_B = 128
_P = 1024
_E = 8
_L = 16


def _im2col(x):
    pos = lax.broadcasted_iota(jnp.int32, (1, _P), 1)
    hh = pos // 32
    ww = pos % 32
    parts = []
    for dh in (-1, 0, 1):
        for dw in (-1, 0, 1):
            s = 32 * dh + dw
            xs = jnp.roll(x, -s, axis=1) if s else x
            conds = []
            if dh == 1:
                conds.append(hh <= 30)
            if dh == -1:
                conds.append(hh >= 1)
            if dw == 1:
                conds.append(ww <= 30)
            if dw == -1:
                conds.append(ww >= 1)
            if conds:
                m = conds[0]
                for c in conds[1:]:
                    m = m & c
                xs = jnp.where(m, xs, jnp.zeros((), xs.dtype))
            parts.append(xs)
    return jnp.concatenate(parts, axis=0)


def _flat_w(w):
    co, ci, _, _ = w.shape
    return w.transpose(0, 2, 3, 1).reshape(co, 9 * ci)


def _split_hl(x):
    hi = x.astype(jnp.bfloat16)
    lo = (x - hi.astype(jnp.float32)).astype(jnp.bfloat16)
    return hi, lo


def _dot3(whi, wlo, xhi, xlo):
    return (jnp.dot(whi, xhi, preferred_element_type=jnp.float32)
            + jnp.dot(whi, xlo, preferred_element_type=jnp.float32)
            + jnp.dot(wlo, xhi, preferred_element_type=jnp.float32))


def _conv_cols(x):
    xhi, xlo = _split_hl(x)
    return _im2col(xhi), _im2col(xlo)


def _make_router(cin, cout, ns=1, interpret=False):
    k9 = 9 * cin

    def body(x_ref, wch_ref, wcl_ref, bc_ref, dw_ref, db_ref, out_ref):
        for j in range(ns):
            xcolh, xcoll = _conv_cols(x_ref[j])
            r = jnp.maximum(
                _dot3(wch_ref[...], wcl_ref[...], xcolh, xcoll)
                + bc_ref[...], 0.0)
            pooled = jnp.sum(r, axis=1, keepdims=True) * (1.0 / _P)
            out_ref[j] = (jnp.sum(dw_ref[...] * pooled, axis=0, keepdims=True)
                          + db_ref[...])

    return pl.pallas_call(
        body,
        grid=(_B // ns,),
        in_specs=[
            pl.BlockSpec((ns, cin, _P), lambda b: (b, 0, 0)),
            pl.BlockSpec((cout, k9), lambda b: (0, 0)),
            pl.BlockSpec((cout, k9), lambda b: (0, 0)),
            pl.BlockSpec((cout, 1), lambda b: (0, 0)),
            pl.BlockSpec((cout, _E), lambda b: (0, 0)),
            pl.BlockSpec((1, _E), lambda b: (0, 0)),
        ],
        out_specs=pl.BlockSpec((ns, 1, _E), lambda b: (b, 0, 0)),
        out_shape=jax.ShapeDtypeStruct((_B, 1, _E), jnp.float32),
        interpret=interpret,
    )


@functools.cache
def _make_route_sc():
    @functools.partial(
        pl.kernel,
        out_type=[
            jax.ShapeDtypeStruct((_B,), jnp.int32),
            jax.ShapeDtypeStruct((_B,), jnp.int32),
            jax.ShapeDtypeStruct((_B,), jnp.float32),
            jax.ShapeDtypeStruct((_B,), jnp.float32),
        ],
        mesh=plsc.VectorSubcoreMesh(core_axis_name="c", subcore_axis_name="s"),
        scratch_types=[
            pltpu.VMEM((_E, _B), jnp.float32),
            pltpu.VMEM((_L,), jnp.int32),
            pltpu.VMEM((_L,), jnp.int32),
            pltpu.VMEM((_L,), jnp.float32),
            pltpu.VMEM((_L,), jnp.float32),
        ])
    def _route_sc(lt_hbm, i0_hbm, i1_hbm, g0_hbm, g1_hbm,
                  lt_v, i0_v, i1_v, g0_v, g1_v):
        n_groups = _B // _L
        wid = lax.axis_index("s") * 2 + lax.axis_index("c")

        @pl.when(wid < n_groups)
        def _():
            pltpu.sync_copy(lt_hbm, lt_v)
            base = wid * _L
            v = [lt_v[e, pl.ds(base, _L)] for e in range(_E)]
            best1 = v[0]
            bi1 = jnp.zeros((_L,), jnp.int32)
            for e in range(1, _E):
                m = v[e] > best1
                best1 = jnp.where(m, v[e], best1)
                bi1 = jnp.where(m, e, bi1)
            best2 = jnp.full((_L,), -3.0e38, jnp.float32)
            bi2 = jnp.zeros((_L,), jnp.int32)
            for e in range(_E):
                m = (bi1 != e) & (v[e] > best2)
                best2 = jnp.where(m, v[e], best2)
                bi2 = jnp.where(m, e, bi2)
            ga = 1.0 / (1.0 + jnp.exp(best2 - best1))
            i0_v[...] = bi1
            i1_v[...] = bi2
            g0_v[...] = ga
            g1_v[...] = 1.0 - ga
            pltpu.sync_copy(i0_v, i0_hbm.at[pl.ds(base, _L)])
            pltpu.sync_copy(i1_v, i1_hbm.at[pl.ds(base, _L)])
            pltpu.sync_copy(g0_v, g0_hbm.at[pl.ds(base, _L)])
            pltpu.sync_copy(g1_v, g1_hbm.at[pl.ds(base, _L)])

    return _route_sc


def _make_experts(cin, cout, tail, ns=1, interpret=False):
    k9 = 9 * cin
    k9h = 9 * cout

    def body(i0_ref, i1_ref, g0_ref, g1_ref, x_ref,
             w1h_ref, w1l_ref, b1_ref, w2h_ref, w2l_ref, b2_ref, *rest):
        if tail == 'head':
            hw_ref, hb_ref = rest[:2]
            out_refs = rest[2:]
        else:
            tah_ref, tal_ref, tb_ref, tc_ref, td_ref = rest[:5]
            out_refs = rest[5:]
        b = pl.program_id(0)
        for j in range(ns):
            s = b * ns + j
            e0 = i0_ref[s]
            e1 = i1_ref[s]
            g0 = g0_ref[s]
            g1 = g1_ref[s]
            xcolh, xcoll = _conv_cols(x_ref[j])
            w1ph = jnp.concatenate([w1h_ref[e0], w1h_ref[e1]], axis=0)
            w1pl = jnp.concatenate([w1l_ref[e0], w1l_ref[e1]], axis=0)
            b1p = jnp.concatenate([b1_ref[e0], b1_ref[e1]], axis=0)
            hh = jnp.maximum(_dot3(w1ph, w1pl, xcolh, xcoll) + b1p, 0.0)
            y = None
            for sl, e, g in ((slice(0, cout), e0, g0),
                             (slice(cout, 2 * cout), e1, g1)):
                hcolh, hcoll = _conv_cols(hh[sl])
                ye = jnp.maximum(
                    _dot3(w2h_ref[e], w2l_ref[e], hcolh, hcoll)
                    + b2_ref[e], 0.0)
                ye = g * ye
                y = ye if y is None else y + ye
            if tail == 'head':
                pooled = jnp.sum(y, axis=1, keepdims=True) * (1.0 / _P)
                out_refs[0][j] = (jnp.sum(hw_ref[...] * pooled, axis=0,
                                          keepdims=True) + hb_ref[...])
            else:
                out_refs[0][j] = y
                ycolh, ycoll = _conv_cols(y)
                r = jnp.maximum(
                    _dot3(tah_ref[...], tal_ref[...], ycolh, ycoll)
                    + tb_ref[...], 0.0)
                pooled = jnp.sum(r, axis=1, keepdims=True) * (1.0 / _P)
                out_refs[1][j] = (jnp.sum(tc_ref[...] * pooled, axis=0,
                                          keepdims=True) + td_ref[...])

    if tail == 'head':
        n_out = 10
        tail_specs = [
            pl.BlockSpec((cout, n_out), lambda b: (0, 0)),
            pl.BlockSpec((1, n_out), lambda b: (0, 0)),
        ]
        out_shape = [jax.ShapeDtypeStruct((_B, 1, n_out), jnp.float32)]
        out_specs = [pl.BlockSpec((ns, 1, n_out), lambda b: (b, 0, 0))]
    else:
        ncout = tail
        tail_specs = [
            pl.BlockSpec((ncout, 9 * cout), lambda b: (0, 0)),
            pl.BlockSpec((ncout, 9 * cout), lambda b: (0, 0)),
            pl.BlockSpec((ncout, 1), lambda b: (0, 0)),
            pl.BlockSpec((ncout, _E), lambda b: (0, 0)),
            pl.BlockSpec((1, _E), lambda b: (0, 0)),
        ]
        out_shape = [jax.ShapeDtypeStruct((_B, cout, _P), jnp.float32),
                     jax.ShapeDtypeStruct((_B, 1, _E), jnp.float32)]
        out_specs = [pl.BlockSpec((ns, cout, _P), lambda b: (b, 0, 0)),
                     pl.BlockSpec((ns, 1, _E), lambda b: (b, 0, 0))]

    return pl.pallas_call(
        body,
        grid=(_B // ns,),
        in_specs=[
            pl.BlockSpec(memory_space=pltpu.SMEM),
            pl.BlockSpec(memory_space=pltpu.SMEM),
            pl.BlockSpec(memory_space=pltpu.SMEM),
            pl.BlockSpec(memory_space=pltpu.SMEM),
            pl.BlockSpec((ns, cin, _P), lambda b: (b, 0, 0)),
            pl.BlockSpec((_E, cout, k9), lambda b: (0, 0, 0)),
            pl.BlockSpec((_E, cout, k9), lambda b: (0, 0, 0)),
            pl.BlockSpec((_E, cout, 1), lambda b: (0, 0, 0)),
            pl.BlockSpec((_E, cout, k9h), lambda b: (0, 0, 0)),
            pl.BlockSpec((_E, cout, k9h), lambda b: (0, 0, 0)),
            pl.BlockSpec((_E, cout, 1), lambda b: (0, 0, 0)),
        ] + tail_specs,
        out_specs=out_specs,
        out_shape=out_shape,
        interpret=interpret,
    )


def _pad8(c):
    return (c + 7) // 8 * 8


def _pad_w(w, co_p, ci_p):
    return jnp.pad(w, ((0, co_p - w.shape[0]), (0, ci_p - w.shape[1]),
                       (0, 0), (0, 0)))


def _pad_rows(a, rows):
    return jnp.pad(a, ((0, rows - a.shape[0]),) + ((0, 0),) * (a.ndim - 1))


def kernel(x, params):
    blocks = params['blocks']
    n = len(blocks)
    cin0 = x.shape[1]
    ci_p = _pad8(cin0)
    h = jnp.pad(x.reshape(_B, cin0, _P), ((0, 0), (0, ci_p - cin0), (0, 0)))
    rt = blocks[0]['router']
    co0_p = _pad8(rt['cw'].shape[0])
    router = _make_router(ci_p, co0_p, ns=8)
    wch, wcl = _split_hl(_flat_w(_pad_w(rt['cw'], co0_p, ci_p)))
    logits = router(h, wch, wcl,
                    _pad_rows(rt['cb'].reshape(-1, 1), co0_p),
                    _pad_rows(rt['dw'], co0_p), rt['db'].reshape(1, _E))
    for bi, blk in enumerate(blocks):
        cout = blk['router']['cw'].shape[0]
        co_p = _pad8(cout)
        i0, i1, g0, g1 = _make_route_sc()(logits.reshape(_B, _E).T)
        w1h, w1l = _split_hl(jnp.stack(
            [_flat_w(_pad_w(ex['w'][0], co_p, ci_p)) for ex in blk['experts']]))
        b1 = jnp.stack([_pad_rows(ex['b'][0].reshape(-1, 1), co_p)
                        for ex in blk['experts']])
        w2h, w2l = _split_hl(jnp.stack(
            [_flat_w(_pad_w(ex['w'][1], co_p, co_p)) for ex in blk['experts']]))
        b2 = jnp.stack([_pad_rows(ex['b'][1].reshape(-1, 1), co_p)
                        for ex in blk['experts']])
        if bi == n - 1:
            experts = _make_experts(ci_p, co_p, tail='head', ns=2)
            out = experts(i0, i1, g0, g1, h, w1h, w1l, b1, w2h, w2l, b2,
                          _pad_rows(params['head']['w'], co_p),
                          params['head']['b'].reshape(1, -1))[0]
            return out.reshape(_B, -1)
        nrt = blocks[bi + 1]['router']
        nco_p = _pad8(nrt['cw'].shape[0])
        nwch, nwcl = _split_hl(_flat_w(_pad_w(nrt['cw'], nco_p, co_p)))
        experts = _make_experts(ci_p, co_p, tail=nco_p,
                                ns=(4 if bi == 0 else 2))
        h, logits = experts(i0, i1, g0, g1, h, w1h, w1l, b1, w2h, w2l, b2,
                            nwch, nwcl,
                            _pad_rows(nrt['cb'].reshape(-1, 1), nco_p),
                            _pad_rows(nrt['dw'], nco_p),
                            nrt['db'].reshape(1, _E))
        ci_p = co_p


# speedup vs baseline: 2.4188x; 1.0020x over previous
import functools

import jax
import jax.numpy as jnp
from jax import lax
from jax.experimental import pallas as pl
from jax.experimental.pallas import tpu as pltpu
from jax.experimental.pallas import tpu_sc as plsc

_B = 128
_P = 1024
_E = 8
_L = 16


def _im2col(x):
    pos = lax.broadcasted_iota(jnp.int32, (1, _P), 1)
    hh = pos // 32
    ww = pos % 32
    zero = jnp.zeros((), x.dtype)
    xw = {
        -1: jnp.where(ww == 31, zero, x),
        0: x,
        1: jnp.where(ww == 0, zero, x),
    }
    parts = []
    for dh in (-1, 0, 1):
        for dw in (-1, 0, 1):
            s = 32 * dh + dw
            xs = jnp.roll(xw[dw], -s, axis=1) if s else x
            if dh == 1:
                xs = jnp.where(hh <= 30, xs, zero)
            elif dh == -1:
                xs = jnp.where(hh >= 1, xs, zero)
            parts.append(xs)
    return jnp.concatenate(parts, axis=0)


def _flat_w(w):
    co, ci, _, _ = w.shape
    return w.transpose(0, 2, 3, 1).reshape(co, 9 * ci)


def _split_hl(x):
    hi = x.astype(jnp.bfloat16)
    lo = (x - hi.astype(jnp.float32)).astype(jnp.bfloat16)
    return hi, lo


def _dot3(whi, wlo, xhi, xlo):
    return (jnp.dot(whi, xhi, preferred_element_type=jnp.float32)
            + jnp.dot(whi, xlo, preferred_element_type=jnp.float32)
            + jnp.dot(wlo, xhi, preferred_element_type=jnp.float32))


def _conv_cols(x):
    xhi, xlo = _split_hl(x)
    return _im2col(xhi), _im2col(xlo)


def _make_router(cin, cout, ns=1, interpret=False):
    k9 = 9 * cin

    def body(x_ref, wch_ref, wcl_ref, bc_ref, dw_ref, db_ref, out_ref):
        for j in range(ns):
            xcolh, xcoll = _conv_cols(x_ref[j])
            r = jnp.maximum(
                _dot3(wch_ref[...], wcl_ref[...], xcolh, xcoll)
                + bc_ref[...], 0.0)
            pooled = jnp.sum(r, axis=1, keepdims=True) * (1.0 / _P)
            out_ref[j] = (jnp.sum(dw_ref[...] * pooled, axis=0, keepdims=True)
                          + db_ref[...])

    return pl.pallas_call(
        body,
        grid=(_B // ns,),
        in_specs=[
            pl.BlockSpec((ns, cin, _P), lambda b: (b, 0, 0)),
            pl.BlockSpec((cout, k9), lambda b: (0, 0)),
            pl.BlockSpec((cout, k9), lambda b: (0, 0)),
            pl.BlockSpec((cout, 1), lambda b: (0, 0)),
            pl.BlockSpec((cout, _E), lambda b: (0, 0)),
            pl.BlockSpec((1, _E), lambda b: (0, 0)),
        ],
        out_specs=pl.BlockSpec((ns, 1, _E), lambda b: (b, 0, 0)),
        out_shape=jax.ShapeDtypeStruct((_B, 1, _E), jnp.float32),
        interpret=interpret,
    )


@functools.cache
def _make_route_sc():
    @functools.partial(
        pl.kernel,
        out_type=[
            jax.ShapeDtypeStruct((_B,), jnp.int32),
            jax.ShapeDtypeStruct((_B,), jnp.int32),
            jax.ShapeDtypeStruct((_B,), jnp.float32),
            jax.ShapeDtypeStruct((_B,), jnp.float32),
        ],
        mesh=plsc.VectorSubcoreMesh(core_axis_name="c", subcore_axis_name="s"),
        scratch_types=[
            pltpu.VMEM((_E, _B), jnp.float32),
            pltpu.VMEM((_L,), jnp.int32),
            pltpu.VMEM((_L,), jnp.int32),
            pltpu.VMEM((_L,), jnp.float32),
            pltpu.VMEM((_L,), jnp.float32),
        ])
    def _route_sc(lt_hbm, i0_hbm, i1_hbm, g0_hbm, g1_hbm,
                  lt_v, i0_v, i1_v, g0_v, g1_v):
        n_groups = _B // _L
        wid = lax.axis_index("s") * 2 + lax.axis_index("c")

        @pl.when(wid < n_groups)
        def _():
            pltpu.sync_copy(lt_hbm, lt_v)
            base = wid * _L
            v = [lt_v[e, pl.ds(base, _L)] for e in range(_E)]
            best1 = v[0]
            bi1 = jnp.zeros((_L,), jnp.int32)
            for e in range(1, _E):
                m = v[e] > best1
                best1 = jnp.where(m, v[e], best1)
                bi1 = jnp.where(m, e, bi1)
            best2 = jnp.full((_L,), -3.0e38, jnp.float32)
            bi2 = jnp.zeros((_L,), jnp.int32)
            for e in range(_E):
                m = (bi1 != e) & (v[e] > best2)
                best2 = jnp.where(m, v[e], best2)
                bi2 = jnp.where(m, e, bi2)
            ga = 1.0 / (1.0 + jnp.exp(best2 - best1))
            i0_v[...] = bi1
            i1_v[...] = bi2
            g0_v[...] = ga
            g1_v[...] = 1.0 - ga
            pltpu.sync_copy(i0_v, i0_hbm.at[pl.ds(base, _L)])
            pltpu.sync_copy(i1_v, i1_hbm.at[pl.ds(base, _L)])
            pltpu.sync_copy(g0_v, g0_hbm.at[pl.ds(base, _L)])
            pltpu.sync_copy(g1_v, g1_hbm.at[pl.ds(base, _L)])

    return _route_sc


def _make_experts(cin, cout, tail, ns=1, interpret=False):
    k9 = 9 * cin
    k9h = 9 * cout

    def body(i0_ref, i1_ref, g0_ref, g1_ref, x_ref,
             w1h_ref, w1l_ref, b1_ref, w2h_ref, w2l_ref, b2_ref, *rest):
        if tail == 'head':
            hw_ref, hb_ref = rest[:2]
            out_refs = rest[2:]
        else:
            tah_ref, tal_ref, tb_ref, tc_ref, td_ref = rest[:5]
            out_refs = rest[5:]
        b = pl.program_id(0)
        for j in range(ns):
            s = b * ns + j
            e0 = i0_ref[s]
            e1 = i1_ref[s]
            g0 = g0_ref[s]
            g1 = g1_ref[s]
            xcolh, xcoll = _conv_cols(x_ref[j])
            w1ph = jnp.concatenate([w1h_ref[e0], w1h_ref[e1]], axis=0)
            w1pl = jnp.concatenate([w1l_ref[e0], w1l_ref[e1]], axis=0)
            b1p = jnp.concatenate([b1_ref[e0], b1_ref[e1]], axis=0)
            hh = jnp.maximum(_dot3(w1ph, w1pl, xcolh, xcoll) + b1p, 0.0)
            y = None
            for sl, e, g in ((slice(0, cout), e0, g0),
                             (slice(cout, 2 * cout), e1, g1)):
                hcolh, hcoll = _conv_cols(hh[sl])
                ye = jnp.maximum(
                    _dot3(w2h_ref[e], w2l_ref[e], hcolh, hcoll)
                    + b2_ref[e], 0.0)
                ye = g * ye
                y = ye if y is None else y + ye
            if tail == 'head':
                pooled = jnp.sum(y, axis=1, keepdims=True) * (1.0 / _P)
                out_refs[0][j] = (jnp.sum(hw_ref[...] * pooled, axis=0,
                                          keepdims=True) + hb_ref[...])
            else:
                out_refs[0][j] = y
                ycolh, ycoll = _conv_cols(y)
                r = jnp.maximum(
                    _dot3(tah_ref[...], tal_ref[...], ycolh, ycoll)
                    + tb_ref[...], 0.0)
                pooled = jnp.sum(r, axis=1, keepdims=True) * (1.0 / _P)
                out_refs[1][j] = (jnp.sum(tc_ref[...] * pooled, axis=0,
                                          keepdims=True) + td_ref[...])

    if tail == 'head':
        n_out = 10
        tail_specs = [
            pl.BlockSpec((cout, n_out), lambda b: (0, 0)),
            pl.BlockSpec((1, n_out), lambda b: (0, 0)),
        ]
        out_shape = [jax.ShapeDtypeStruct((_B, 1, n_out), jnp.float32)]
        out_specs = [pl.BlockSpec((ns, 1, n_out), lambda b: (b, 0, 0))]
    else:
        ncout = tail
        tail_specs = [
            pl.BlockSpec((ncout, 9 * cout), lambda b: (0, 0)),
            pl.BlockSpec((ncout, 9 * cout), lambda b: (0, 0)),
            pl.BlockSpec((ncout, 1), lambda b: (0, 0)),
            pl.BlockSpec((ncout, _E), lambda b: (0, 0)),
            pl.BlockSpec((1, _E), lambda b: (0, 0)),
        ]
        out_shape = [jax.ShapeDtypeStruct((_B, cout, _P), jnp.float32),
                     jax.ShapeDtypeStruct((_B, 1, _E), jnp.float32)]
        out_specs = [pl.BlockSpec((ns, cout, _P), lambda b: (b, 0, 0)),
                     pl.BlockSpec((ns, 1, _E), lambda b: (b, 0, 0))]

    return pl.pallas_call(
        body,
        grid=(_B // ns,),
        in_specs=[
            pl.BlockSpec(memory_space=pltpu.SMEM),
            pl.BlockSpec(memory_space=pltpu.SMEM),
            pl.BlockSpec(memory_space=pltpu.SMEM),
            pl.BlockSpec(memory_space=pltpu.SMEM),
            pl.BlockSpec((ns, cin, _P), lambda b: (b, 0, 0)),
            pl.BlockSpec((_E, cout, k9), lambda b: (0, 0, 0)),
            pl.BlockSpec((_E, cout, k9), lambda b: (0, 0, 0)),
            pl.BlockSpec((_E, cout, 1), lambda b: (0, 0, 0)),
            pl.BlockSpec((_E, cout, k9h), lambda b: (0, 0, 0)),
            pl.BlockSpec((_E, cout, k9h), lambda b: (0, 0, 0)),
            pl.BlockSpec((_E, cout, 1), lambda b: (0, 0, 0)),
        ] + tail_specs,
        out_specs=out_specs,
        out_shape=out_shape,
        interpret=interpret,
    )


def _pad8(c):
    return (c + 7) // 8 * 8


def _pad_w(w, co_p, ci_p):
    return jnp.pad(w, ((0, co_p - w.shape[0]), (0, ci_p - w.shape[1]),
                       (0, 0), (0, 0)))


def _pad_rows(a, rows):
    return jnp.pad(a, ((0, rows - a.shape[0]),) + ((0, 0),) * (a.ndim - 1))


def kernel(x, params):
    blocks = params['blocks']
    n = len(blocks)
    cin0 = x.shape[1]
    ci_p = _pad8(cin0)
    h = jnp.pad(x.reshape(_B, cin0, _P), ((0, 0), (0, ci_p - cin0), (0, 0)))
    rt = blocks[0]['router']
    co0_p = _pad8(rt['cw'].shape[0])
    router = _make_router(ci_p, co0_p, ns=8)
    wch, wcl = _split_hl(_flat_w(_pad_w(rt['cw'], co0_p, ci_p)))
    logits = router(h, wch, wcl,
                    _pad_rows(rt['cb'].reshape(-1, 1), co0_p),
                    _pad_rows(rt['dw'], co0_p), rt['db'].reshape(1, _E))
    for bi, blk in enumerate(blocks):
        cout = blk['router']['cw'].shape[0]
        co_p = _pad8(cout)
        i0, i1, g0, g1 = _make_route_sc()(logits.reshape(_B, _E).T)
        w1h, w1l = _split_hl(jnp.stack(
            [_flat_w(_pad_w(ex['w'][0], co_p, ci_p)) for ex in blk['experts']]))
        b1 = jnp.stack([_pad_rows(ex['b'][0].reshape(-1, 1), co_p)
                        for ex in blk['experts']])
        w2h, w2l = _split_hl(jnp.stack(
            [_flat_w(_pad_w(ex['w'][1], co_p, co_p)) for ex in blk['experts']]))
        b2 = jnp.stack([_pad_rows(ex['b'][1].reshape(-1, 1), co_p)
                        for ex in blk['experts']])
        if bi == n - 1:
            experts = _make_experts(ci_p, co_p, tail='head', ns=2)
            out = experts(i0, i1, g0, g1, h, w1h, w1l, b1, w2h, w2l, b2,
                          _pad_rows(params['head']['w'], co_p),
                          params['head']['b'].reshape(1, -1))[0]
            return out.reshape(_B, -1)
        nrt = blocks[bi + 1]['router']
        nco_p = _pad8(nrt['cw'].shape[0])
        nwch, nwcl = _split_hl(_flat_w(_pad_w(nrt['cw'], nco_p, co_p)))
        experts = _make_experts(ci_p, co_p, tail=nco_p,
                                ns=(4 if bi == 0 else 2))
        h, logits = experts(i0, i1, g0, g1, h, w1h, w1l, b1, w2h, w2l, b2,
                            nwch, nwcl,
                            _pad_rows(nrt['cb'].reshape(-1, 1), nco_p),
                            _pad_rows(nrt['dw'], nco_p),
                            nrt['db'].reshape(1, _E))
        ci_p = co_p
